# baseline (speedup 1.0000x reference)
import functools

import jax
import jax.numpy as jnp
from jax import lax
from jax.experimental import pallas as pl
from jax.experimental.pallas import tpu as pltpu
from jax.experimental.pallas import tpu_sc as plsc

N = 10000
D = 128
E = 320000

NC, NS = 2, 16
NW = NC * NS
NPAD = 10240
RPT = NPAD // NS
DEGW = 16

CH = 125
GD = E // (NW * CH)
GE = E // (NS * CH)
QD = D // 4
NBUF = 4


def _sc_deg_body(dst_hbm, deg_out, idx_v, ones_v, zer_v, deg_sh):
    c = lax.axis_index("c")
    s = lax.axis_index("s")
    pltpu.sync_copy(dst_hbm.at[1, s, pl.ds(c * GD, GD)], idx_v)

    def fill_ones(r, _):
        ones_v[r, :] = jnp.ones((DEGW,), jnp.float32)
        return 0

    lax.fori_loop(0, CH, fill_ones, 0)

    def fill_zer(r, _):
        zer_v[r, :] = jnp.zeros((DEGW,), jnp.float32)
        return 0

    lax.fori_loop(0, RPT, fill_zer, 0)
    pltpu.sync_copy(zer_v, deg_sh.at[pl.ds(s * RPT, RPT)])
    plsc.subcore_barrier()

    def add_chunk(g, _):
        pltpu.sync_copy(ones_v, deg_sh.at[idx_v.at[g]], add=True)
        return 0

    lax.fori_loop(0, GD, add_chunk, 0)
    plsc.subcore_barrier()
    pltpu.sync_copy(deg_sh.at[pl.ds(s * RPT, RPT)],
                    deg_out.at[c, pl.ds(s * RPT, RPT)])


def _sc_edge_body(hp_hbm, ev_hbm, acc_out, src_v, dst_v, zbuf, bufs,
                  table_sh, acc_sh, gsems, ssems):
    c = lax.axis_index("c")
    s = lax.axis_index("s")
    pltpu.sync_copy(ev_hbm.at[0, s], src_v)
    pltpu.sync_copy(ev_hbm.at[1, s], dst_v)

    def fz(r, _):
        for k in range(QD // 16):
            zbuf[r, pl.ds(k * 16, 16)] = jnp.zeros((16,), jnp.float32)
        return 0

    lax.fori_loop(0, zbuf.shape[0], fz, 0)

    for r in range(2):
        qc = 2 * r + c
        pltpu.sync_copy(hp_hbm.at[pl.ds(s * RPT, RPT), pl.ds(qc * QD, QD)],
                        table_sh.at[pl.ds(s * RPT, RPT)])
        for k in range(RPT // zbuf.shape[0]):
            pltpu.sync_copy(
                zbuf, acc_sh.at[pl.ds(s * RPT + k * zbuf.shape[0],
                                      zbuf.shape[0])])
        plsc.subcore_barrier()

        def fire_gather(g, j):
            pltpu.async_copy(table_sh.at[src_v.at[g]], bufs[j], gsems[j])

        def wait_gather(g, j):
            pltpu.make_async_copy(table_sh.at[src_v.at[g]], bufs[j],
                                  gsems[j]).wait()

        def fire_scatter(g, j):
            pltpu.async_copy(bufs[j], acc_sh.at[dst_v.at[g]], ssems[j],
                             add=True)

        def wait_scatter(g, j):
            pltpu.make_async_copy(bufs[j], acc_sh.at[dst_v.at[g]],
                                  ssems[j]).wait()

        NB2 = 2 * NBUF
        P = GE // NB2

        for j in range(NB2):
            fire_gather(j, j)

        def pair(p, fire_next):
            for half in range(2):
                base = NB2 * p + NBUF * half
                js = range(NBUF * half, NBUF * half + NBUF)
                for i, j in enumerate(js):
                    wait_gather(base + i, j)
                for i, j in enumerate(js):
                    fire_scatter(base + i, j)
                for i, j in enumerate(js):
                    wait_scatter(base + i, j)
                if fire_next:
                    for i, j in enumerate(js):
                        fire_gather(base + NB2 + i, j)

        def outer(p, _):
            pair(p, True)
            return 0

        lax.fori_loop(0, P - 1, outer, 0)
        pair(P - 1, False)

        plsc.subcore_barrier()
        pltpu.sync_copy(acc_sh.at[pl.ds(s * RPT, RPT)],
                        acc_out.at[pl.ds(s * RPT, RPT), pl.ds(qc * QD, QD)])


@functools.lru_cache(maxsize=None)
def _sc_kernels():
    mesh = plsc.VectorSubcoreMesh(
        core_axis_name="c", subcore_axis_name="s",
        num_cores=NC, num_subcores=NS)
    params = pltpu.CompilerParams(use_tc_tiling_on_sc=False)
    sc_deg = pl.kernel(
        _sc_deg_body,
        out_type=jax.ShapeDtypeStruct((NC, NPAD, DEGW), jnp.float32),
        mesh=mesh,
        compiler_params=params,
        scratch_types=[
            pltpu.VMEM((GD, CH), jnp.int32),
            pltpu.VMEM((CH, DEGW), jnp.float32),
            pltpu.VMEM((RPT, DEGW), jnp.float32),
            pltpu.VMEM_SHARED((NPAD, DEGW), jnp.float32),
        ],
    )
    sc_edge = pl.kernel(
        _sc_edge_body,
        out_type=jax.ShapeDtypeStruct((NPAD, D), jnp.float32),
        mesh=mesh,
        compiler_params=params,
        scratch_types=[
            pltpu.VMEM((GE, CH), jnp.int32),
            pltpu.VMEM((GE, CH), jnp.int32),
            pltpu.VMEM((128, QD), jnp.float32),
            [pltpu.VMEM((CH, QD), jnp.float32) for _ in range(2 * NBUF)],
            pltpu.VMEM_SHARED((NPAD, QD), jnp.float32),
            pltpu.VMEM_SHARED((NPAD, QD), jnp.float32),
            [pltpu.SemaphoreType.DMA for _ in range(2 * NBUF)],
            [pltpu.SemaphoreType.DMA for _ in range(2 * NBUF)],
        ],
    )
    return sc_deg, sc_edge


BLK = 1000
_GRID = N // BLK


def _row_spec():
    return pl.BlockSpec((BLK, D), lambda i: (i, 0))


def _deg_spec():
    return pl.BlockSpec((BLK, DEGW), lambda i: (i, 0))


def _full_spec(r):
    return pl.BlockSpec(r, lambda i: (0, 0))


def _dinv(dega_ref, degb_ref):
    deg = dega_ref[:, 0:1] + degb_ref[:, 0:1] + 1.0
    return lax.rsqrt(deg)


def _tc_mm_body(x_ref, w_ref, out_ref):
    out_ref[...] = jnp.dot(x_ref[...], w_ref[...],
                           preferred_element_type=jnp.float32)


_tc_mm = pl.pallas_call(
    _tc_mm_body,
    grid=(_GRID,),
    in_specs=[_row_spec(), _full_spec((D, D))],
    out_specs=_row_spec(),
    out_shape=jax.ShapeDtypeStruct((NPAD, D), jnp.float32),
)


def _tc_scale_body(h_ref, dega_ref, degb_ref, out_ref):
    out_ref[...] = h_ref[...] * _dinv(dega_ref, degb_ref)


_tc_scale = pl.pallas_call(
    _tc_scale_body,
    grid=(_GRID,),
    in_specs=[_row_spec(), _deg_spec(), _deg_spec()],
    out_specs=_row_spec(),
    out_shape=jax.ShapeDtypeStruct((NPAD, D), jnp.float32),
)


def _tc_mid_body(acc_ref, hp_ref, dega_ref, degb_ref, b_ref, w_ref, out_ref):
    dinv = _dinv(dega_ref, degb_ref)
    t = dinv * (acc_ref[...] + hp_ref[...]) + b_ref[...]
    t = jnp.maximum(t, 0.0)
    h = jnp.dot(t, w_ref[...], preferred_element_type=jnp.float32)
    out_ref[...] = h * dinv


_tc_mid = pl.pallas_call(
    _tc_mid_body,
    grid=(_GRID,),
    in_specs=[_row_spec(), _row_spec(), _deg_spec(), _deg_spec(),
              _full_spec((1, D)), _full_spec((D, D))],
    out_specs=_row_spec(),
    out_shape=jax.ShapeDtypeStruct((NPAD, D), jnp.float32),
)


def _tc_post_body(acc_ref, hp_ref, dega_ref, degb_ref, b_ref, out_ref):
    dinv = _dinv(dega_ref, degb_ref)
    out_ref[...] = dinv * (acc_ref[...] + hp_ref[...]) + b_ref[...]


_tc_post = pl.pallas_call(
    _tc_post_body,
    grid=(_GRID,),
    in_specs=[_row_spec(), _row_spec(), _deg_spec(), _deg_spec(),
              _full_spec((1, D))],
    out_specs=_row_spec(),
    out_shape=jax.ShapeDtypeStruct((N, D), jnp.float32),
)


def kernel(x, edge_index, W1, b1, W2, b2):
    ev = edge_index.reshape(2, NS, GE, CH)
    b1r = b1.reshape(1, D)
    b2r = b2.reshape(1, D)

    sc_deg, sc_edge = _sc_kernels()
    deg = sc_deg(ev)
    h1 = _tc_mm(x, W1)
    dega, degb = deg[0], deg[1]
    hp1 = _tc_scale(h1, dega, degb)
    acc1 = sc_edge(hp1, ev)
    hp2 = _tc_mid(acc1, hp1, dega, degb, b1r, W2)
    acc2 = sc_edge(hp2, ev)
    out = _tc_post(acc2, hp2, dega, degb, b2r)
    return out


# speedup vs baseline: 1.0209x; 1.0209x over previous
import functools

import jax
import jax.numpy as jnp
from jax import lax
from jax.experimental import pallas as pl
from jax.experimental.pallas import tpu as pltpu
from jax.experimental.pallas import tpu_sc as plsc

N = 10000
D = 128
E = 320000

NC, NS = 2, 16
NW = NC * NS
NPAD = 10240
RPT = NPAD // NS
DEGW = 16

CH = 125
GD = E // (NW * CH)
GE = E // (NS * CH)
QD = D // 4
NBUF = 4


def _sc_deg_body(dst_hbm, deg_out, idx_v, ones_v, zer_v, deg_sh):
    c = lax.axis_index("c")
    s = lax.axis_index("s")
    pltpu.sync_copy(dst_hbm.at[1, s, pl.ds(c * GD, GD)], idx_v)

    def fill_ones(r, _):
        ones_v[r, :] = jnp.ones((DEGW,), jnp.float32)
        return 0

    lax.fori_loop(0, CH, fill_ones, 0)

    def fill_zer(r, _):
        zer_v[r, :] = jnp.zeros((DEGW,), jnp.float32)
        return 0

    lax.fori_loop(0, RPT, fill_zer, 0)
    pltpu.sync_copy(zer_v, deg_sh.at[pl.ds(s * RPT, RPT)])
    plsc.subcore_barrier()

    def add_chunk(g, _):
        pltpu.sync_copy(ones_v, deg_sh.at[idx_v.at[g]], add=True)
        return 0

    lax.fori_loop(0, GD, add_chunk, 0)
    plsc.subcore_barrier()
    pltpu.sync_copy(deg_sh.at[pl.ds(s * RPT, RPT)],
                    deg_out.at[c, pl.ds(s * RPT, RPT)])


def _sc_edge_body(hp_hbm, ev_hbm, acc_out, src_v, dst_v, zbuf, bufs,
                  table_sh, acc_sh, gsems, ssems):
    c = lax.axis_index("c")
    s = lax.axis_index("s")
    pltpu.async_copy(ev_hbm.at[0, s], src_v, gsems[6])
    pltpu.async_copy(ev_hbm.at[1, s], dst_v, gsems[7])

    def fz(r, _):
        for k in range(QD // 16):
            zbuf[r, pl.ds(k * 16, 16)] = jnp.zeros((16,), jnp.float32)
        return 0

    lax.fori_loop(0, zbuf.shape[0], fz, 0)
    pltpu.make_async_copy(ev_hbm.at[0, s], src_v, gsems[6]).wait()
    pltpu.make_async_copy(ev_hbm.at[1, s], dst_v, gsems[7]).wait()

    ZR = zbuf.shape[0]

    for r in range(2):
        qc = 2 * r + c
        pltpu.async_copy(hp_hbm.at[pl.ds(s * RPT, RPT),
                                   pl.ds(qc * QD, QD)],
                         table_sh.at[pl.ds(s * RPT, RPT)], gsems[5])
        for k in range(RPT // ZR):
            pltpu.async_copy(zbuf, acc_sh.at[pl.ds(s * RPT + k * ZR, ZR)],
                             gsems[k])
        pltpu.make_async_copy(hp_hbm.at[pl.ds(s * RPT, RPT),
                                        pl.ds(qc * QD, QD)],
                              table_sh.at[pl.ds(s * RPT, RPT)],
                              gsems[5]).wait()
        for k in range(RPT // ZR):
            pltpu.make_async_copy(zbuf,
                                  acc_sh.at[pl.ds(s * RPT + k * ZR, ZR)],
                                  gsems[k]).wait()
        plsc.subcore_barrier()

        def fire_gather(g, j):
            pltpu.async_copy(table_sh.at[src_v.at[g]], bufs[j], gsems[j])

        def wait_gather(g, j):
            pltpu.make_async_copy(table_sh.at[src_v.at[g]], bufs[j],
                                  gsems[j]).wait()

        def fire_scatter(g, j):
            pltpu.async_copy(bufs[j], acc_sh.at[dst_v.at[g]], ssems[j],
                             add=True)

        def wait_scatter(g, j):
            pltpu.make_async_copy(bufs[j], acc_sh.at[dst_v.at[g]],
                                  ssems[j]).wait()

        NB2 = 2 * NBUF
        P = GE // NB2

        for j in range(NB2):
            fire_gather(j, j)

        def pair(p, fire_next):
            for half in range(2):
                base = NB2 * p + NBUF * half
                js = range(NBUF * half, NBUF * half + NBUF)
                for i, j in enumerate(js):
                    wait_gather(base + i, j)
                for i, j in enumerate(js):
                    fire_scatter(base + i, j)
                for i, j in enumerate(js):
                    wait_scatter(base + i, j)
                if fire_next:
                    for i, j in enumerate(js):
                        fire_gather(base + NB2 + i, j)

        def outer(p, _):
            pair(p, True)
            return 0

        lax.fori_loop(0, P - 1, outer, 0)
        pair(P - 1, False)

        plsc.subcore_barrier()
        pltpu.sync_copy(acc_sh.at[pl.ds(s * RPT, RPT)],
                        acc_out.at[pl.ds(s * RPT, RPT), pl.ds(qc * QD, QD)])


@functools.lru_cache(maxsize=None)
def _sc_kernels():
    mesh = plsc.VectorSubcoreMesh(
        core_axis_name="c", subcore_axis_name="s",
        num_cores=NC, num_subcores=NS)
    params = pltpu.CompilerParams(use_tc_tiling_on_sc=False)
    sc_deg = pl.kernel(
        _sc_deg_body,
        out_type=jax.ShapeDtypeStruct((NC, NPAD, DEGW), jnp.float32),
        mesh=mesh,
        compiler_params=params,
        scratch_types=[
            pltpu.VMEM((GD, CH), jnp.int32),
            pltpu.VMEM((CH, DEGW), jnp.float32),
            pltpu.VMEM((RPT, DEGW), jnp.float32),
            pltpu.VMEM_SHARED((NPAD, DEGW), jnp.float32),
        ],
    )
    sc_edge = pl.kernel(
        _sc_edge_body,
        out_type=jax.ShapeDtypeStruct((NPAD, D), jnp.float32),
        mesh=mesh,
        compiler_params=params,
        scratch_types=[
            pltpu.VMEM((GE, CH), jnp.int32),
            pltpu.VMEM((GE, CH), jnp.int32),
            pltpu.VMEM((128, QD), jnp.float32),
            [pltpu.VMEM((CH, QD), jnp.float32) for _ in range(2 * NBUF)],
            pltpu.VMEM_SHARED((NPAD, QD), jnp.float32),
            pltpu.VMEM_SHARED((NPAD, QD), jnp.float32),
            [pltpu.SemaphoreType.DMA for _ in range(2 * NBUF)],
            [pltpu.SemaphoreType.DMA for _ in range(2 * NBUF)],
        ],
    )
    return sc_deg, sc_edge


BLK = 1000
_GRID = N // BLK


def _row_spec():
    return pl.BlockSpec((BLK, D), lambda i: (i, 0))


def _deg_spec():
    return pl.BlockSpec((BLK, DEGW), lambda i: (i, 0))


def _full_spec(r):
    return pl.BlockSpec(r, lambda i: (0, 0))


def _dinv(dega_ref, degb_ref):
    deg = dega_ref[:, 0:1] + degb_ref[:, 0:1] + 1.0
    return lax.rsqrt(deg)


def _tc_pre_body(x_ref, w_ref, dega_ref, degb_ref, out_ref):
    dinv = _dinv(dega_ref, degb_ref)
    h = jnp.dot(x_ref[...], w_ref[...], preferred_element_type=jnp.float32)
    out_ref[...] = h * dinv


_tc_pre = pl.pallas_call(
    _tc_pre_body,
    grid=(_GRID,),
    in_specs=[_row_spec(), _full_spec((D, D)), _deg_spec(), _deg_spec()],
    out_specs=_row_spec(),
    out_shape=jax.ShapeDtypeStruct((NPAD, D), jnp.float32),
)


def _tc_mid_body(acc_ref, hp_ref, dega_ref, degb_ref, b_ref, w_ref, out_ref):
    dinv = _dinv(dega_ref, degb_ref)
    t = dinv * (acc_ref[...] + hp_ref[...]) + b_ref[...]
    t = jnp.maximum(t, 0.0)
    h = jnp.dot(t, w_ref[...], preferred_element_type=jnp.float32)
    out_ref[...] = h * dinv


_tc_mid = pl.pallas_call(
    _tc_mid_body,
    grid=(_GRID,),
    in_specs=[_row_spec(), _row_spec(), _deg_spec(), _deg_spec(),
              _full_spec((1, D)), _full_spec((D, D))],
    out_specs=_row_spec(),
    out_shape=jax.ShapeDtypeStruct((NPAD, D), jnp.float32),
)


def _tc_post_body(acc_ref, hp_ref, dega_ref, degb_ref, b_ref, out_ref):
    dinv = _dinv(dega_ref, degb_ref)
    out_ref[...] = dinv * (acc_ref[...] + hp_ref[...]) + b_ref[...]


_tc_post = pl.pallas_call(
    _tc_post_body,
    grid=(_GRID,),
    in_specs=[_row_spec(), _row_spec(), _deg_spec(), _deg_spec(),
              _full_spec((1, D))],
    out_specs=_row_spec(),
    out_shape=jax.ShapeDtypeStruct((N, D), jnp.float32),
)


def kernel(x, edge_index, W1, b1, W2, b2):
    ev = edge_index.reshape(2, NS, GE, CH)
    b1r = b1.reshape(1, D)
    b2r = b2.reshape(1, D)

    sc_deg, sc_edge = _sc_kernels()
    deg = sc_deg(ev)
    dega, degb = deg[0], deg[1]
    hp1 = _tc_pre(x, W1, dega, degb)
    acc1 = sc_edge(hp1, ev)
    hp2 = _tc_mid(acc1, hp1, dega, degb, b1r, W2)
    acc2 = sc_edge(hp2, ev)
    out = _tc_post(acc2, hp2, dega, degb, b2r)
    return out
